# baseline (speedup 1.0000x reference)
import jax
import jax.numpy as jnp
from jax.experimental import pallas as pl
from jax.experimental.pallas import tpu as pltpu

B = 2048
D = 128
E = 64
R = 256
K = 8
G = 4
NSTEP = E // G


def _gate_kernel(h_ref, u_ref, U_ref, V_ref, bias_ref, out_ref,
                 hb_ref, vb_ref, g_ref):
    i = pl.program_id(0)
    prev = ((i + 1) % 2) * (G * R)
    cur = (i % 2) * (G * R)
    row0 = jnp.maximum(i - 1, 0) * G

    hUT = hb_ref[pl.ds(prev, G * R), :]
    uVT = vb_ref[pl.ds(prev, G * R), :]
    p = hUT * uVT
    for j in range(G):
        pj = p[j * R:(j + 1) * R, :]
        g_ref[pl.ds(row0 + j, 1), :] = jnp.sum(pj, axis=0, keepdims=True)

    hb_ref[pl.ds(cur, G * R), :] = jax.lax.dot_general(
        U_ref[...], h_ref[...], (((1,), (1,)), ((), ())),
        preferred_element_type=jnp.float32)
    vb_ref[pl.ds(cur, G * R), :] = jax.lax.dot_general(
        V_ref[...], u_ref[...], (((1,), (1,)), ((), ())),
        preferred_element_type=jnp.float32)

    @pl.when(i == NSTEP - 1)
    def _():
        hUT = hb_ref[pl.ds(cur, G * R), :]
        uVT = vb_ref[pl.ds(cur, G * R), :]
        p = hUT * uVT
        for j in range(G):
            pj = p[j * R:(j + 1) * R, :]
            g_ref[pl.ds(i * G + j, 1), :] = jnp.sum(pj, axis=0,
                                                    keepdims=True)
        x = g_ref[...] + bias_ref[...]
        rem = x
        for _ in range(K - 1):
            m = jnp.max(rem, axis=0, keepdims=True)
            rem = jnp.where(rem >= m, -jnp.inf, rem)
        t8 = jnp.max(rem, axis=0, keepdims=True)
        sel = x >= t8
        xm = jnp.max(x, axis=0, keepdims=True)
        ex = jnp.where(sel, jnp.exp(x - xm), 0.0)
        w = ex / jnp.sum(ex, axis=0, keepdims=True)
        out_ref[...] = jax.lax.transpose(w, (1, 0))


def kernel(h, u, U, V, bias):
    Ur = U.reshape(E * R, D)
    Vr = V.reshape(E * R, D)
    bias2 = bias.reshape(E, 1)
    return pl.pallas_call(
        _gate_kernel,
        grid=(NSTEP,),
        in_specs=[
            pl.BlockSpec((B, D), lambda i: (0, 0)),
            pl.BlockSpec((B, D), lambda i: (0, 0)),
            pl.BlockSpec((G * R, D), lambda i: (i, 0)),
            pl.BlockSpec((G * R, D), lambda i: (i, 0)),
            pl.BlockSpec((E, 1), lambda i: (0, 0)),
        ],
        out_specs=pl.BlockSpec((B, E), lambda i: (0, 0)),
        out_shape=jax.ShapeDtypeStruct((B, E), jnp.float32),
        scratch_shapes=[pltpu.VMEM((2 * G * R, B), jnp.float32),
                        pltpu.VMEM((2 * G * R, B), jnp.float32),
                        pltpu.VMEM((E, B), jnp.float32)],
    )(h, u, Ur, Vr, bias2)


# speedup vs baseline: 1.4149x; 1.4149x over previous
import jax
import jax.numpy as jnp
from jax.experimental import pallas as pl
from jax.experimental.pallas import tpu as pltpu

B = 2048
D = 128
E = 64
R = 256
K = 8
G = 4
NSTEP = E // G


def _gate_kernel(h_ref, u_ref, U_ref, V_ref, bias_ref, out_ref,
                 hA_ref, vA_ref, hB_ref, vB_ref, g_ref):
    i = pl.program_id(0)
    row0 = jnp.maximum(i - 1, 0) * G

    def drain(hb_ref, vb_ref, r0):
        p = hb_ref[...] * vb_ref[...]
        for j in range(G):
            pj = p[j * R:(j + 1) * R, :]
            g_ref[pl.ds(r0 + j, 1), :] = jnp.sum(pj, axis=0, keepdims=True)

    def fill(hb_ref, vb_ref):
        hb_ref[...] = jax.lax.dot_general(
            U_ref[...], h_ref[...], (((1,), (1,)), ((), ())),
            preferred_element_type=jnp.float32)
        vb_ref[...] = jax.lax.dot_general(
            V_ref[...], u_ref[...], (((1,), (1,)), ((), ())),
            preferred_element_type=jnp.float32)

    @pl.when(i % 2 == 0)
    def _():
        drain(hB_ref, vB_ref, row0)
        fill(hA_ref, vA_ref)

    @pl.when(i % 2 == 1)
    def _():
        drain(hA_ref, vA_ref, row0)
        fill(hB_ref, vB_ref)

    @pl.when(i == NSTEP - 1)
    def _():
        drain(hB_ref, vB_ref, (NSTEP - 1) * G)
        x = g_ref[...] + bias_ref[...]
        rem = x
        for _ in range(K - 1):
            m = jnp.max(rem, axis=0, keepdims=True)
            rem = jnp.where(rem >= m, -jnp.inf, rem)
        t8 = jnp.max(rem, axis=0, keepdims=True)
        sel = x >= t8
        xm = jnp.max(x, axis=0, keepdims=True)
        ex = jnp.where(sel, jnp.exp(x - xm), 0.0)
        w = ex / jnp.sum(ex, axis=0, keepdims=True)
        out_ref[...] = jax.lax.transpose(w, (1, 0))


def kernel(h, u, U, V, bias):
    Ur = U.reshape(E * R, D)
    Vr = V.reshape(E * R, D)
    bias2 = bias.reshape(E, 1)
    return pl.pallas_call(
        _gate_kernel,
        grid=(NSTEP,),
        in_specs=[
            pl.BlockSpec((B, D), lambda i: (0, 0)),
            pl.BlockSpec((B, D), lambda i: (0, 0)),
            pl.BlockSpec((G * R, D), lambda i: (i, 0)),
            pl.BlockSpec((G * R, D), lambda i: (i, 0)),
            pl.BlockSpec((E, 1), lambda i: (0, 0)),
        ],
        out_specs=pl.BlockSpec((B, E), lambda i: (0, 0)),
        out_shape=jax.ShapeDtypeStruct((B, E), jnp.float32),
        scratch_shapes=[pltpu.VMEM((G * R, B), jnp.float32),
                        pltpu.VMEM((G * R, B), jnp.float32),
                        pltpu.VMEM((G * R, B), jnp.float32),
                        pltpu.VMEM((G * R, B), jnp.float32),
                        pltpu.VMEM((E, B), jnp.float32)],
    )(h, u, Ur, Vr, bias2)


# speedup vs baseline: 2.0259x; 1.4318x over previous
import jax
import jax.numpy as jnp
from jax.experimental import pallas as pl
from jax.experimental.pallas import tpu as pltpu

B = 2048
D = 128
E = 64
R = 256
K = 8
G = 4
C = 2


def _gate_kernel(h_ref, u_ref, U_ref, V_ref, bias_ref, out_ref, g_ref):
    i = pl.program_id(0)
    h = h_ref[...]
    u = u_ref[...]

    for c in range(G // C):
        lo = c * C * R
        Uc = U_ref[lo:lo + C * R, :]
        Vc = V_ref[lo:lo + C * R, :]
        hUT = jax.lax.dot_general(Uc, h, (((1,), (1,)), ((), ())),
                                  preferred_element_type=jnp.float32)
        uVT = jax.lax.dot_general(Vc, u, (((1,), (1,)), ((), ())),
                                  preferred_element_type=jnp.float32)
        p = hUT * uVT
        for j in range(C):
            pj = p[j * R:(j + 1) * R, :]
            g_ref[pl.ds(i * G + c * C + j, 1), :] = jnp.sum(
                pj, axis=0, keepdims=True)

    @pl.when(i == (E // G) - 1)
    def _():
        x = g_ref[...] + bias_ref[...]
        rem = x
        for _ in range(K - 1):
            m = jnp.max(rem, axis=0, keepdims=True)
            rem = jnp.where(rem >= m, -jnp.inf, rem)
        t8 = jnp.max(rem, axis=0, keepdims=True)
        sel = x >= t8
        xm = jnp.max(x, axis=0, keepdims=True)
        ex = jnp.where(sel, jnp.exp(x - xm), 0.0)
        w = ex / jnp.sum(ex, axis=0, keepdims=True)
        out_ref[...] = jax.lax.transpose(w, (1, 0))


def kernel(h, u, U, V, bias):
    Ur = U.reshape(E * R, D)
    Vr = V.reshape(E * R, D)
    bias2 = bias.reshape(E, 1)
    return pl.pallas_call(
        _gate_kernel,
        grid=(E // G,),
        in_specs=[
            pl.BlockSpec((B, D), lambda i: (0, 0)),
            pl.BlockSpec((B, D), lambda i: (0, 0)),
            pl.BlockSpec((G * R, D), lambda i: (i, 0)),
            pl.BlockSpec((G * R, D), lambda i: (i, 0)),
            pl.BlockSpec((E, 1), lambda i: (0, 0)),
        ],
        out_specs=pl.BlockSpec((B, E), lambda i: (0, 0)),
        out_shape=jax.ShapeDtypeStruct((B, E), jnp.float32),
        scratch_shapes=[pltpu.VMEM((E, B), jnp.float32)],
    )(h, u, Ur, Vr, bias2)
